# baseline (speedup 1.0000x reference)
import functools

import jax
import jax.numpy as jnp
from jax import lax
from jax.experimental import pallas as pl
from jax.experimental.pallas import tpu as pltpu
from jax.experimental.pallas import tpu_sc as plsc

NUM_EMBEDDINGS = 100001
EMBED_DIM = 32
BATCH = 16384

_G = 16
_NBUF = 4
_ROW = EMBED_DIM


def _make_gather():
    info = plsc.get_sparse_core_info()
    nc, ns = info.num_cores, info.num_subcores
    nw = nc * ns
    b_per_w = BATCH // nw
    n_groups = b_per_w // _G
    gw = _G * _ROW
    mesh = plsc.VectorSubcoreMesh(core_axis_name="c", subcore_axis_name="s")

    @functools.partial(
        pl.kernel,
        mesh=mesh,
        out_type=jax.ShapeDtypeStruct((BATCH * _ROW,), jnp.float32),
        scratch_types=[
            pltpu.VMEM((b_per_w,), jnp.int32),
            pltpu.VMEM((b_per_w * _ROW,), jnp.float32),
            pltpu.SemaphoreType.DMA,
            pltpu.SemaphoreType.DMA,
        ],
    )
    def gather_kernel(table_hbm, idx_hbm, out_hbm, idx_v, rows_v, gsem, wsem):
        wid = lax.axis_index("s") * nc + lax.axis_index("c")
        base = wid * b_per_w
        pltpu.sync_copy(idx_hbm.at[pl.ds(base, b_per_w)], idx_v)

        def enqueue(g):
            vec = idx_v[pl.ds(g * _G, _G)]
            for j in range(_G):
                pltpu.async_copy(
                    table_hbm.at[pl.ds(vec[j] * _ROW, _ROW)],
                    rows_v.at[pl.ds((g * _G + j) * _ROW, _ROW)],
                    gsem,
                )

        def finish(g):
            pltpu.make_async_copy(
                table_hbm.at[pl.ds(0, gw)],
                rows_v.at[pl.ds(g * gw, gw)],
                gsem,
            ).wait()
            pltpu.async_copy(
                rows_v.at[pl.ds(g * gw, gw)],
                out_hbm.at[pl.ds(base * _ROW + g * gw, gw)],
                wsem,
            )

        for g in range(_NBUF):
            enqueue(g)

        def step(g, carry):
            enqueue(g)
            finish(g - _NBUF)
            return carry

        lax.fori_loop(_NBUF, n_groups, step, 0)
        for g in range(n_groups - _NBUF, n_groups):
            finish(g)
        pltpu.make_async_copy(
            rows_v, out_hbm.at[pl.ds(base * _ROW, b_per_w * _ROW)], wsem
        ).wait()

    return gather_kernel


def kernel(titles, embedding_table):
    gather_kernel = _make_gather()
    out_flat = gather_kernel(embedding_table.reshape(-1), titles)
    return out_flat.reshape(BATCH, EMBED_DIM)


# speedup vs baseline: 2.1634x; 2.1634x over previous
import functools

import jax
import jax.numpy as jnp
from jax import lax
from jax.experimental import pallas as pl
from jax.experimental.pallas import tpu as pltpu
from jax.experimental.pallas import tpu_sc as plsc

NUM_EMBEDDINGS = 100001
EMBED_DIM = 32
BATCH = 16384

_CH = 2048


def _make_gather():
    info = plsc.get_sparse_core_info()
    nc, ns, nl = info.num_cores, info.num_subcores, info.num_lanes
    nw = nc * ns
    n_chunks = BATCH // _CH
    mesh = plsc.VectorSubcoreMesh(core_axis_name="c", subcore_axis_name="s")

    @functools.partial(
        pl.kernel,
        mesh=mesh,
        compiler_params=pltpu.CompilerParams(needs_layout_passes=False),
        out_type=jax.ShapeDtypeStruct((EMBED_DIM, BATCH), jnp.float32),
        scratch_types=[
            pltpu.VMEM((NUM_EMBEDDINGS,), jnp.float32),
            pltpu.VMEM((_CH,), jnp.int32),
            pltpu.VMEM((_CH,), jnp.float32),
        ],
    )
    def gather_kernel(table_hbm, idx_hbm, out_hbm, col_v, tit_v, res_v):
        wid = lax.axis_index("s") * nc + lax.axis_index("c")
        pltpu.sync_copy(table_hbm.at[wid], col_v)

        for t in range(n_chunks):
            pltpu.sync_copy(idx_hbm.at[pl.ds(t * _CH, _CH)], tit_v)

            def gstep(i, carry):
                idx = tit_v[pl.ds(i * nl, nl)]
                res_v[pl.ds(i * nl, nl)] = plsc.load_gather(col_v, [idx])
                return carry

            lax.fori_loop(0, _CH // nl, gstep, 0)
            pltpu.sync_copy(res_v, out_hbm.at[wid, pl.ds(t * _CH, _CH)])

    return gather_kernel


def kernel(titles, embedding_table):
    gather_kernel = _make_gather()
    out_t = gather_kernel(embedding_table.T, titles)
    return out_t.T


# speedup vs baseline: 2.3785x; 1.0994x over previous
import functools

import jax
import jax.numpy as jnp
from jax import lax
from jax.experimental import pallas as pl
from jax.experimental.pallas import tpu as pltpu
from jax.experimental.pallas import tpu_sc as plsc

NUM_EMBEDDINGS = 100001
EMBED_DIM = 32
BATCH = 16384

_CH = 4096


def _make_gather():
    info = plsc.get_sparse_core_info()
    nc, ns, nl = info.num_cores, info.num_subcores, info.num_lanes
    nw = nc * ns
    n_chunks = BATCH // _CH
    mesh = plsc.VectorSubcoreMesh(core_axis_name="c", subcore_axis_name="s")

    @functools.partial(
        pl.kernel,
        mesh=mesh,
        compiler_params=pltpu.CompilerParams(needs_layout_passes=False),
        out_type=jax.ShapeDtypeStruct((EMBED_DIM, BATCH), jnp.float32),
        scratch_types=[
            pltpu.VMEM((NUM_EMBEDDINGS,), jnp.float32),
            pltpu.VMEM((BATCH,), jnp.int32),
            pltpu.VMEM((2, _CH), jnp.float32),
            pltpu.SemaphoreType.DMA,
            pltpu.SemaphoreType.DMA,
            pltpu.SemaphoreType.DMA,
        ],
    )
    def gather_kernel(
        table_hbm, idx_hbm, out_hbm, col_v, tit_v, res_v, csem, wsem0, wsem1
    ):
        wid = lax.axis_index("s") * nc + lax.axis_index("c")
        stage_col = pltpu.async_copy(table_hbm.at[wid], col_v, csem)
        stage_idx = pltpu.async_copy(idx_hbm, tit_v, csem)
        stage_col.wait()
        stage_idx.wait()

        wsems = (wsem0, wsem1)
        for t in range(n_chunks):
            buf = t % 2
            if t >= 2:
                pltpu.make_async_copy(
                    res_v.at[buf],
                    out_hbm.at[wid, pl.ds((t - 2) * _CH, _CH)],
                    wsems[buf],
                ).wait()

            def gstep(i, carry, t=t, buf=buf):
                idx = tit_v[pl.ds(t * _CH + i * nl, nl)]
                res_v[buf, pl.ds(i * nl, nl)] = plsc.load_gather(col_v, [idx])
                return carry

            lax.fori_loop(0, _CH // nl, gstep, 0)
            pltpu.async_copy(
                res_v.at[buf],
                out_hbm.at[wid, pl.ds(t * _CH, _CH)],
                wsems[buf],
            )
        for t in (n_chunks - 2, n_chunks - 1):
            buf = t % 2
            pltpu.make_async_copy(
                res_v.at[buf],
                out_hbm.at[wid, pl.ds(t * _CH, _CH)],
                wsems[buf],
            ).wait()

    return gather_kernel


def kernel(titles, embedding_table):
    gather_kernel = _make_gather()
    out_t = gather_kernel(embedding_table.T, titles)
    return out_t.T
